# baseline (speedup 1.0000x reference)
import functools

import jax
import jax.numpy as jnp
from jax import lax
from jax.experimental import pallas as pl
from jax.experimental.pallas import tpu as pltpu
from jax.experimental.pallas import tpu_sc as plsc

N_NODES = 10000
N_EDGES = 320000
N_GRAPHS = 64
IN_DIM = 128
HID_DIM = 64
OUT_DIM = 64
LAYERS = 4
EPS_BN = 1e-5


_NC = 2
_NS = 16
_NW = _NC * _NS
_CHUNK = 125
_NCHUNKS = N_EDGES // _CHUNK
_CH_PER_W = _NCHUNKS // _NW
_ROWS_PER_TILE = 632
_N_PAD = _ROWS_PER_TILE * _NS


_DEPTH = 8


def _sc_edge_agg(h, edges3d, zrows):
    mesh = plsc.VectorSubcoreMesh(core_axis_name="c", subcore_axis_name="s")

    @functools.partial(
        pl.kernel,
        mesh=mesh,
        out_type=jax.ShapeDtypeStruct((_NC, _N_PAD, HID_DIM), jnp.float32),
        scratch_types=[
            pltpu.VMEM((_CH_PER_W, _CHUNK), jnp.int32),
            pltpu.VMEM((_CH_PER_W, _CHUNK), jnp.int32),
            [pltpu.VMEM((_CHUNK, HID_DIM), jnp.float32) for _ in range(_DEPTH)],
            [pltpu.SemaphoreType.DMA for _ in range(_DEPTH)],
            [pltpu.SemaphoreType.DMA for _ in range(_DEPTH)],
            pltpu.VMEM_SHARED((_N_PAD, HID_DIM), jnp.float32),
        ],
        compiler_params=pltpu.CompilerParams(use_tc_tiling_on_sc=False),
    )
    def agg_kernel(h_hbm, e_hbm, z_hbm, out_hbm,
                   src_v, dst_v, rows, gsems, ssems, acc_sh):
        c = lax.axis_index("c")
        s = lax.axis_index("s")
        w = s * _NC + c

        r0 = s * _ROWS_PER_TILE
        zslab = acc_sh.at[pl.ds(r0, _ROWS_PER_TILE)]
        pltpu.async_copy(z_hbm, zslab, ssems[_DEPTH - 1])
        pltpu.sync_copy(e_hbm.at[0, pl.ds(w * _CH_PER_W, _CH_PER_W)], src_v)
        pltpu.sync_copy(e_hbm.at[1, pl.ds(w * _CH_PER_W, _CH_PER_W)], dst_v)

        def gstart(chunk, p):
            pltpu.async_copy(h_hbm.at[src_v.at[chunk]], rows[p], gsems[p])

        def gwait(chunk, p):
            pltpu.make_async_copy(h_hbm.at[src_v.at[chunk]], rows[p],
                                  gsems[p]).wait()

        def sstart(chunk, p):
            pltpu.async_copy(rows[p], acc_sh.at[dst_v.at[chunk]], ssems[p],
                             add=True)

        def swait(chunk, p):
            pltpu.make_async_copy(rows[p], acc_sh.at[dst_v.at[chunk]],
                                  ssems[p]).wait()

        for cc in range(_DEPTH):
            gstart(cc, cc)
        pltpu.make_async_copy(z_hbm, zslab, ssems[_DEPTH - 1]).wait()
        plsc.subcore_barrier()
        _AHEAD = _DEPTH // 2
        for cc in range(_AHEAD):
            gwait(cc, cc)
            sstart(cc, cc)

        def body(j, carry):
            c0 = _AHEAD + _DEPTH * j
            for k in range(_DEPTH):
                ck = c0 + k
                p = (_AHEAD + k) % _DEPTH
                q = k
                swait(ck - _AHEAD, q)
                gstart(ck + _AHEAD, q)
                gwait(ck, p)
                sstart(ck, p)
            return carry

        n_steady = _CH_PER_W - 2 * _AHEAD
        lax.fori_loop(0, n_steady // _DEPTH, body, 0)
        for ck in range(_CH_PER_W - _AHEAD, _CH_PER_W):
            gwait(ck, ck % _DEPTH)
            sstart(ck, ck % _DEPTH)
        for ck in range(_CH_PER_W - _DEPTH, _CH_PER_W):
            swait(ck, ck % _DEPTH)

        plsc.subcore_barrier()
        pltpu.sync_copy(acc_sh.at[pl.ds(r0, _ROWS_PER_TILE)],
                        out_hbm.at[c, pl.ds(r0, _ROWS_PER_TILE)])

    return agg_kernel(h, edges3d, zrows)


_PREC = lax.Precision.DEFAULT


_NP = N_NODES // 2
_PACK = 2 * HID_DIM


_HI = lax.Precision.HIGHEST


def _bn_relu_packed(y, fold_ref, unfold_ref, g2, be2):
    m = jnp.dot(jnp.mean(y, axis=0, keepdims=True), fold_ref[...],
                precision=_HI) * 0.5
    yc = y - jnp.dot(m, unfold_ref[...], precision=_HI)
    v = jnp.dot(jnp.mean(yc * yc, axis=0, keepdims=True), fold_ref[...],
                precision=_HI) * 0.5
    vb = jnp.dot(v, unfold_ref[...], precision=_HI)
    return jnp.maximum(yc * lax.rsqrt(vb + EPS_BN) * g2 + be2, 0.0)


def _first_body(x_ref, fold_ref, unfold_ref, w1t_ref, b1_ref, g1_ref,
                be1_ref, w2t_ref, b2_ref, g2_ref, be2_ref, h_ref):
    y = jnp.dot(x_ref[...], w1t_ref[...], precision=_PREC) + b1_ref[...]
    y = _bn_relu_packed(y, fold_ref, unfold_ref, g1_ref[...], be1_ref[...])
    y = jnp.dot(y, w2t_ref[...], precision=_PREC) + b2_ref[...]
    h_ref[...] = _bn_relu_packed(y, fold_ref, unfold_ref, g2_ref[...],
                                 be2_ref[...])


def _layer_body(h_in_ref, agg_ref, eps_ref, fold_ref, unfold_ref, w1t_ref,
                b1_ref, g1_ref, be1_ref, w2t_ref, b2_ref, g2_ref, be2_ref,
                h_ref):
    u = (h_in_ref[...] * (1.0 + eps_ref[...])
         + agg_ref[0, :_NP, :] + agg_ref[1, :_NP, :])
    y = jnp.dot(u, w1t_ref[...], precision=_PREC) + b1_ref[...]
    y = _bn_relu_packed(y, fold_ref, unfold_ref, g1_ref[...], be1_ref[...])
    y = jnp.dot(y, w2t_ref[...], precision=_PREC) + b2_ref[...]
    h_ref[...] = _bn_relu_packed(y, fold_ref, unfold_ref, g2_ref[...],
                                 be2_ref[...])


def _onehots(be_ref, bo_ref):
    ae = (lax.broadcasted_iota(jnp.int32, (N_GRAPHS, _NP), 0)
          == be_ref[...]).astype(jnp.float32)
    ao = (lax.broadcasted_iota(jnp.int32, (N_GRAPHS, _NP), 0)
          == bo_ref[...]).astype(jnp.float32)
    cnt = (jnp.sum(ae, axis=1, keepdims=True)
           + jnp.sum(ao, axis=1, keepdims=True))
    return ae, ao, 1.0 / jnp.maximum(cnt, 1.0)


def _seg_mean(ae, ao, inv_cnt, hp, se_ref, so_ref):
    sums = (jnp.dot(jnp.dot(ae, hp, precision=_PREC), se_ref[...],
                    precision=_PREC)
            + jnp.dot(jnp.dot(ao, hp, precision=_PREC), so_ref[...],
                      precision=_PREC))
    return sums * inv_cnt


def _poolpre_body(be_ref, bo_ref, se_ref, so_ref, h1_ref, h2_ref, h3_ref,
                  wlts_ref, blsum_ref, out_ref):
    ae, ao, inv_cnt = _onehots(be_ref, bo_ref)
    acc = blsum_ref[...]
    for i, h_ref in enumerate((h1_ref, h2_ref, h3_ref)):
        pooled = _seg_mean(ae, ao, inv_cnt, h_ref[...], se_ref, so_ref)
        acc = acc + jnp.dot(pooled, wlts_ref[i], precision=_PREC)
    out_ref[...] = acc


def _poolfin_body(be_ref, bo_ref, se_ref, so_ref, h4_ref, w4t_ref, pacc_ref,
                  out_ref):
    ae, ao, inv_cnt = _onehots(be_ref, bo_ref)
    pooled = _seg_mean(ae, ao, inv_cnt, h4_ref[...], se_ref, so_ref)
    out_ref[...] = pacc_ref[...] + jnp.dot(pooled, w4t_ref[...],
                                           precision=_PREC)


def _poolall_body(be_ref, bo_ref, se_ref, so_ref, h1_ref, h2_ref, h3_ref,
                  h4_ref, wlts_ref, blsum_ref, out_ref):
    ae, ao, inv_cnt = _onehots(be_ref, bo_ref)
    acc = blsum_ref[...]
    for i, h_ref in enumerate((h1_ref, h2_ref, h3_ref, h4_ref)):
        pooled = _seg_mean(ae, ao, inv_cnt, h_ref[...], se_ref, so_ref)
        acc = acc + jnp.dot(pooled, wlts_ref[i], precision=_PREC)
    out_ref[...] = acc


_H_OUT = jax.ShapeDtypeStruct((_NP, _PACK), jnp.float32)
_OUT_SD = jax.ShapeDtypeStruct((N_GRAPHS, OUT_DIM), jnp.float32)

_first_call = pl.pallas_call(_first_body, out_shape=_H_OUT)
_layer_call = pl.pallas_call(_layer_body, out_shape=_H_OUT)
_poolpre_call = pl.pallas_call(_poolpre_body, out_shape=_OUT_SD)
_poolfin_call = pl.pallas_call(_poolfin_body, out_shape=_OUT_SD)
_poolall_call = pl.pallas_call(_poolall_body, out_shape=_OUT_SD)


def _blockdiag(w):
    z = jnp.zeros_like(w)
    return jnp.block([[w, z], [z, w]])


def _tile2(v):
    return jnp.concatenate([v, v]).reshape(1, -1)


def _mlp_args(p):
    return (_blockdiag(p["W1"].T), _tile2(p["b1"]), _tile2(p["g1"]),
            _tile2(p["be1"]), _blockdiag(p["W2"].T), _tile2(p["b2"]),
            _tile2(p["g2"]), _tile2(p["be2"]))


def kernel(x, edge_index, batch, params):
    edges3d = edge_index.reshape(2, _NCHUNKS, _CHUNK)
    zrows = jnp.zeros((_ROWS_PER_TILE, HID_DIM), jnp.float32)

    eye = jnp.eye(HID_DIM, dtype=jnp.float32)
    zed = jnp.zeros((HID_DIM, HID_DIM), jnp.float32)
    fold = jnp.concatenate([eye, eye], axis=0)
    unfold = jnp.concatenate([eye, eye], axis=1)
    se = jnp.concatenate([eye, zed], axis=0)
    so = jnp.concatenate([zed, eye], axis=0)

    x_p = x.reshape(_NP, 2 * IN_DIM)
    b2d = batch.reshape(_NP, 2)
    b_even = b2d[:, 0].reshape(1, _NP)
    b_odd = b2d[:, 1].reshape(1, _NP)

    lin = params["lin"]
    hs = [_first_call(x_p, fold, unfold, *_mlp_args(params["first_h"]))]
    for layer in range(1, LAYERS):
        agg = _sc_edge_agg(hs[-1].reshape(N_NODES, HID_DIM), edges3d, zrows)
        agg_p = agg.reshape(_NC, _N_PAD // 2, _PACK)
        eps = params["eps"][layer - 1].reshape(1, 1)
        hs.append(_layer_call(hs[-1], agg_p, eps, fold, unfold,
                              *_mlp_args(params["nns"][layer - 1])))

    wlts = jnp.stack([lin[i]["W"].T for i in range(LAYERS)])
    blsum = sum(lin[i]["b"] for i in range(LAYERS)).reshape(1, -1)
    return _poolall_call(b_even, b_odd, se, so, *hs, wlts, blsum)


# speedup vs baseline: 1.0119x; 1.0119x over previous
import functools

import jax
import jax.numpy as jnp
from jax import lax
from jax.experimental import pallas as pl
from jax.experimental.pallas import tpu as pltpu
from jax.experimental.pallas import tpu_sc as plsc

N_NODES = 10000
N_EDGES = 320000
N_GRAPHS = 64
IN_DIM = 128
HID_DIM = 64
OUT_DIM = 64
LAYERS = 4
EPS_BN = 1e-5


_NC = 2
_NS = 16
_NW = _NC * _NS
_CHUNK = 125
_NCHUNKS = N_EDGES // _CHUNK
_CH_PER_W = _NCHUNKS // _NW
_ROWS_PER_TILE = 632
_N_PAD = _ROWS_PER_TILE * _NS


_DEPTH = 8


def _sc_edge_agg(h, edges3d, zrows):
    mesh = plsc.VectorSubcoreMesh(core_axis_name="c", subcore_axis_name="s")

    @functools.partial(
        pl.kernel,
        mesh=mesh,
        out_type=jax.ShapeDtypeStruct((_NC, _N_PAD, HID_DIM), jnp.float32),
        scratch_types=[
            pltpu.VMEM((_CH_PER_W, _CHUNK), jnp.int32),
            pltpu.VMEM((_CH_PER_W, _CHUNK), jnp.int32),
            [pltpu.VMEM((_CHUNK, HID_DIM), jnp.float32) for _ in range(_DEPTH)],
            [pltpu.SemaphoreType.DMA for _ in range(_DEPTH)],
            [pltpu.SemaphoreType.DMA for _ in range(_DEPTH)],
            pltpu.VMEM_SHARED((_N_PAD, HID_DIM), jnp.float32),
        ],
        compiler_params=pltpu.CompilerParams(use_tc_tiling_on_sc=False),
    )
    def agg_kernel(h_hbm, e_hbm, z_hbm, out_hbm,
                   src_v, dst_v, rows, gsems, ssems, acc_sh):
        c = lax.axis_index("c")
        s = lax.axis_index("s")
        w = s * _NC + c

        r0 = s * _ROWS_PER_TILE
        zslab = acc_sh.at[pl.ds(r0, _ROWS_PER_TILE)]
        pltpu.async_copy(z_hbm, zslab, ssems[_DEPTH - 1])
        pltpu.sync_copy(e_hbm.at[0, pl.ds(w * _CH_PER_W, _CH_PER_W)], src_v)
        pltpu.sync_copy(e_hbm.at[1, pl.ds(w * _CH_PER_W, _CH_PER_W)], dst_v)

        def gstart(chunk, p):
            pltpu.async_copy(h_hbm.at[src_v.at[chunk]], rows[p], gsems[p])

        def gwait(chunk, p):
            pltpu.make_async_copy(h_hbm.at[src_v.at[chunk]], rows[p],
                                  gsems[p]).wait()

        def sstart(chunk, p):
            pltpu.async_copy(rows[p], acc_sh.at[dst_v.at[chunk]], ssems[p],
                             add=True)

        def swait(chunk, p):
            pltpu.make_async_copy(rows[p], acc_sh.at[dst_v.at[chunk]],
                                  ssems[p]).wait()

        for cc in range(_DEPTH):
            gstart(cc, cc)
        pltpu.make_async_copy(z_hbm, zslab, ssems[_DEPTH - 1]).wait()
        plsc.subcore_barrier()
        _AHEAD = _DEPTH // 2
        for cc in range(_AHEAD):
            gwait(cc, cc)
            sstart(cc, cc)

        def body(j, carry):
            c0 = _AHEAD + _DEPTH * j
            for k in range(_DEPTH):
                ck = c0 + k
                p = (_AHEAD + k) % _DEPTH
                q = k
                swait(ck - _AHEAD, q)
                gstart(ck + _AHEAD, q)
                gwait(ck, p)
                sstart(ck, p)
            return carry

        n_steady = _CH_PER_W - 2 * _AHEAD
        lax.fori_loop(0, n_steady // _DEPTH, body, 0)
        for ck in range(_CH_PER_W - _AHEAD, _CH_PER_W):
            gwait(ck, ck % _DEPTH)
            sstart(ck, ck % _DEPTH)
        for ck in range(_CH_PER_W - _DEPTH, _CH_PER_W):
            swait(ck, ck % _DEPTH)

        plsc.subcore_barrier()
        pltpu.sync_copy(acc_sh.at[pl.ds(r0, _ROWS_PER_TILE)],
                        out_hbm.at[c, pl.ds(r0, _ROWS_PER_TILE)])

    return agg_kernel(h, edges3d, zrows)


_PREC = lax.Precision.DEFAULT


_NP = N_NODES // 2
_PACK = 2 * HID_DIM


_HI = lax.Precision.HIGHEST


def _bn_relu_packed(y, fold_ref, unfold_ref, g2, be2):
    m = jnp.dot(jnp.mean(y, axis=0, keepdims=True), fold_ref[...],
                precision=_HI) * 0.5
    yc = y - jnp.dot(m, unfold_ref[...], precision=_HI)
    v = jnp.dot(jnp.mean(yc * yc, axis=0, keepdims=True), fold_ref[...],
                precision=_HI) * 0.5
    vb = jnp.dot(v, unfold_ref[...], precision=_HI)
    return jnp.maximum(yc * lax.rsqrt(vb + EPS_BN) * g2 + be2, 0.0)


def _first_body(x_ref, fold_ref, unfold_ref, w1t_ref, b1_ref, g1_ref,
                be1_ref, w2t_ref, b2_ref, g2_ref, be2_ref, h_ref):
    y = jnp.dot(x_ref[...], w1t_ref[...], precision=_PREC) + b1_ref[...]
    y = _bn_relu_packed(y, fold_ref, unfold_ref, g1_ref[...], be1_ref[...])
    y = jnp.dot(y, w2t_ref[...], precision=_PREC) + b2_ref[...]
    h_ref[...] = _bn_relu_packed(y, fold_ref, unfold_ref, g2_ref[...],
                                 be2_ref[...])


def _layer_body(h_in_ref, agg_ref, eps_ref, fold_ref, unfold_ref, w1t_ref,
                b1_ref, g1_ref, be1_ref, w2t_ref, b2_ref, g2_ref, be2_ref,
                h_ref):
    u = (h_in_ref[...] * (1.0 + eps_ref[...])
         + agg_ref[0, :_NP, :] + agg_ref[1, :_NP, :])
    y = jnp.dot(u, w1t_ref[...], precision=_PREC) + b1_ref[...]
    y = _bn_relu_packed(y, fold_ref, unfold_ref, g1_ref[...], be1_ref[...])
    y = jnp.dot(y, w2t_ref[...], precision=_PREC) + b2_ref[...]
    h_ref[...] = _bn_relu_packed(y, fold_ref, unfold_ref, g2_ref[...],
                                 be2_ref[...])


def _onehots(be_ref, bo_ref):
    ae = (lax.broadcasted_iota(jnp.int32, (N_GRAPHS, _NP), 0)
          == be_ref[...]).astype(jnp.float32)
    ao = (lax.broadcasted_iota(jnp.int32, (N_GRAPHS, _NP), 0)
          == bo_ref[...]).astype(jnp.float32)
    cnt = (jnp.sum(ae, axis=1, keepdims=True)
           + jnp.sum(ao, axis=1, keepdims=True))
    return ae, ao, 1.0 / jnp.maximum(cnt, 1.0)


def _seg_mean(ae, ao, inv_cnt, hp, se_ref, so_ref):
    sums = (jnp.dot(jnp.dot(ae, hp, precision=_PREC), se_ref[...],
                    precision=_PREC)
            + jnp.dot(jnp.dot(ao, hp, precision=_PREC), so_ref[...],
                      precision=_PREC))
    return sums * inv_cnt


def _poolpre_body(be_ref, bo_ref, se_ref, so_ref, h1_ref, h2_ref, h3_ref,
                  wlts_ref, blsum_ref, out_ref):
    ae, ao, inv_cnt = _onehots(be_ref, bo_ref)
    acc = blsum_ref[...]
    for i, h_ref in enumerate((h1_ref, h2_ref, h3_ref)):
        pooled = _seg_mean(ae, ao, inv_cnt, h_ref[...], se_ref, so_ref)
        acc = acc + jnp.dot(pooled, wlts_ref[i], precision=_PREC)
    out_ref[...] = acc


def _poolfin_body(be_ref, bo_ref, se_ref, so_ref, h4_ref, w4t_ref, pacc_ref,
                  out_ref):
    ae, ao, inv_cnt = _onehots(be_ref, bo_ref)
    pooled = _seg_mean(ae, ao, inv_cnt, h4_ref[...], se_ref, so_ref)
    out_ref[...] = pacc_ref[...] + jnp.dot(pooled, w4t_ref[...],
                                           precision=_PREC)


_H_OUT = jax.ShapeDtypeStruct((_NP, _PACK), jnp.float32)
_OUT_SD = jax.ShapeDtypeStruct((N_GRAPHS, OUT_DIM), jnp.float32)

_first_call = pl.pallas_call(_first_body, out_shape=_H_OUT)
_layer_call = pl.pallas_call(_layer_body, out_shape=_H_OUT)
_poolpre_call = pl.pallas_call(_poolpre_body, out_shape=_OUT_SD)
_poolfin_call = pl.pallas_call(_poolfin_body, out_shape=_OUT_SD)


def _blockdiag(w):
    z = jnp.zeros_like(w)
    return jnp.block([[w, z], [z, w]])


def _tile2(v):
    return jnp.concatenate([v, v]).reshape(1, -1)


def _mlp_args(p):
    return (_blockdiag(p["W1"].T), _tile2(p["b1"]), _tile2(p["g1"]),
            _tile2(p["be1"]), _blockdiag(p["W2"].T), _tile2(p["b2"]),
            _tile2(p["g2"]), _tile2(p["be2"]))


def kernel(x, edge_index, batch, params):
    edges3d = edge_index.reshape(2, _NCHUNKS, _CHUNK)
    zrows = jnp.zeros((_ROWS_PER_TILE, HID_DIM), jnp.float32)

    eye = jnp.eye(HID_DIM, dtype=jnp.float32)
    zed = jnp.zeros((HID_DIM, HID_DIM), jnp.float32)
    fold = jnp.concatenate([eye, eye], axis=0)
    unfold = jnp.concatenate([eye, eye], axis=1)
    se = jnp.concatenate([eye, zed], axis=0)
    so = jnp.concatenate([zed, eye], axis=0)

    x_p = x.reshape(_NP, 2 * IN_DIM)
    b2d = batch.reshape(_NP, 2)
    b_even = b2d[:, 0].reshape(1, _NP)
    b_odd = b2d[:, 1].reshape(1, _NP)

    lin = params["lin"]
    hs = [_first_call(x_p, fold, unfold, *_mlp_args(params["first_h"]))]
    for layer in range(1, LAYERS):
        agg = _sc_edge_agg(hs[-1].reshape(N_NODES, HID_DIM), edges3d, zrows)
        agg_p = agg.reshape(_NC, _N_PAD // 2, _PACK)
        eps = params["eps"][layer - 1].reshape(1, 1)
        hs.append(_layer_call(hs[-1], agg_p, eps, fold, unfold,
                              *_mlp_args(params["nns"][layer - 1])))

    wlts3 = jnp.stack([lin[i]["W"].T for i in range(LAYERS - 1)])
    blsum = sum(lin[i]["b"] for i in range(LAYERS)).reshape(1, -1)
    pacc = _poolpre_call(b_even, b_odd, se, so, hs[0], hs[1], hs[2],
                         wlts3, blsum)
    return _poolfin_call(b_even, b_odd, se, so, hs[3], lin[3]["W"].T, pacc)
